# Initial kernel scaffold; baseline (speedup 1.0000x reference)
import jax
import jax.numpy as jnp
from jax.experimental import pallas as pl


def kernel(query, reference_points_cam, bev_mask):
    raise NotImplementedError("write your pallas kernel here")


# speedup vs baseline: 6.5682x; 6.5682x over previous
import jax
import jax.numpy as jnp
from jax.experimental import pallas as pl

MAXLEN = 8
BIG = 2 ** 30


def _msda_body(q_ref, bm_ref, rp_ref, o_ref):
    hits = bm_ref[0] + bm_ref[1] + bm_ref[2] + bm_ref[3]
    hit = hits > 0
    hit_f = hit.astype(jnp.float32)

    count = jnp.sum(hit_f, axis=0, keepdims=True)
    cnorm = 1.0 / jnp.maximum(count, 1.0)

    n = hit.shape[1]
    iota = jax.lax.broadcasted_iota(jnp.int32, hit.shape, 1)
    masked = jnp.where(hit, iota, BIG)
    thresh = None
    for _ in range(MAXLEN):
        thresh = jnp.min(masked, axis=1, keepdims=True)
        masked = jnp.where(masked == thresh, BIG, masked)
    sel = hit_f * (iota <= thresh).astype(jnp.float32)

    rsum = rp_ref[0]
    for p in range(1, 8):
        rsum = rsum + rp_ref[p]
    attn = jnp.tanh(rsum * 0.125)

    s = jnp.sum(sel * (1.0 + attn), axis=0, keepdims=True) * cnorm
    o_ref[...] = q_ref[...] * s.T


def kernel(query, reference_points_cam, bev_mask):
    _, n, d = query.shape
    q = query[0]
    bm = jnp.transpose(bev_mask[:, 0], (2, 0, 1))
    rp = jnp.transpose(
        reference_points_cam[:, 0].reshape(bev_mask.shape[0], n, 8), (2, 0, 1)
    )
    out = pl.pallas_call(
        _msda_body,
        out_shape=jax.ShapeDtypeStruct((n, d), jnp.float32),
    )(q, bm, rp)
    return out[None]
